# Initial kernel scaffold; baseline (speedup 1.0000x reference)
import functools

import jax
import jax.numpy as jnp
from jax import lax
from jax.experimental import pallas as pl
from jax.experimental.pallas import tpu as pltpu
from jax.experimental.pallas import tpu_sc as plsc

_SEC_PER_YEAR = 3600.0 * 24.0 * 365.0
_SEC_PER_MONTH = 3600.0 * 24.0 * 30.0


@functools.partial(jax.jit, static_argnames=("B", "L", "DE", "TD"))
def _run(table, idx, ts, W_t, b_t, B, L, DE, TD):
    info = plsc.get_sparse_core_info()
    NC, NS = info.num_cores, info.num_subcores
    NW = NC * NS
    D = DE + TD
    NB = B // NW
    C0, C1 = 104, L - 104

    mesh = plsc.VectorSubcoreMesh(core_axis_name="c", subcore_axis_name="s")

    @functools.partial(
        pl.kernel,
        mesh=mesh,
        out_type=jax.ShapeDtypeStruct((B * L, D), jnp.float32),
        scratch_types=[
            pltpu.VMEM((L,), jnp.int32),
            pltpu.VMEM((L,), jnp.float32),
            pltpu.VMEM((L, DE), jnp.float32),
            pltpu.VMEM((L, TD), jnp.float32),
            pltpu.VMEM((2, TD), jnp.float32),
            pltpu.VMEM((TD,), jnp.float32),
            pltpu.SemaphoreType.DMA,
        ],
    )
    def k(table_hbm, idx_hbm, ts_hbm, w_hbm, bt_hbm, out_hbm,
          idx_v, ts_v, emb_v, te_v, w_v, bt_v, gsem):
        wid = lax.axis_index("s") * NC + lax.axis_index("c")
        pltpu.sync_copy(w_hbm, w_v)
        pltpu.sync_copy(bt_hbm, bt_v)
        w0 = w_v[0, :]
        w1 = w_v[1, :]
        bt = bt_v[...]

        def b_body(i, carry):
            base = (wid * NB + i) * L
            pltpu.sync_copy(idx_hbm.at[pl.ds(base, L)], idx_v)
            pltpu.sync_copy(ts_hbm.at[pl.ds(base, L)], ts_v)
            g0 = pltpu.async_copy(
                table_hbm.at[idx_v.at[pl.ds(0, C0)]],
                emb_v.at[pl.ds(0, C0)], gsem)
            g1 = pltpu.async_copy(
                table_hbm.at[idx_v.at[pl.ds(C0, C1)]],
                emb_v.at[pl.ds(C0, C1)], gsem)
            t0 = ts_v[0]

            def te_body(r, carry2):
                rr = r * 8
                for u in range(8):
                    tt = ts_v[rr + u]
                    a = tt * (1.0 / _SEC_PER_YEAR)
                    rl = (tt - t0) * (1.0 / _SEC_PER_MONTH)
                    te_v[rr + u, :] = jnp.maximum(a * w0 + rl * w1 + bt, 0.0)
                return carry2

            lax.fori_loop(0, L // 8, te_body, 0)
            g0.wait()
            g1.wait()
            pltpu.sync_copy(emb_v, out_hbm.at[pl.ds(base, L), pl.ds(0, DE)])
            pltpu.sync_copy(te_v, out_hbm.at[pl.ds(base, L), pl.ds(DE, TD)])
            return carry

        lax.fori_loop(0, NB, b_body, 0)

    return k(table, idx, ts, W_t, b_t)


def kernel(indices, timestamps, table, W_t, b_t):
    B, L = indices.shape
    DE = table.shape[1]
    TD = b_t.shape[0]
    idx = indices.reshape(-1).astype(jnp.int32)
    ts = timestamps.reshape(-1)
    out = _run(table, idx, ts, W_t, b_t, B, L, DE, TD)
    return out.reshape(B, L, DE + TD)


# speedup vs baseline: 4.6070x; 4.6070x over previous
import functools

import jax
import jax.numpy as jnp
from jax import lax
from jax.experimental import pallas as pl
from jax.experimental.pallas import tpu as pltpu
from jax.experimental.pallas import tpu_sc as plsc

_SEC_PER_YEAR = 3600.0 * 24.0 * 365.0
_SEC_PER_MONTH = 3600.0 * 24.0 * 30.0


@functools.partial(jax.jit, static_argnames=("B", "L", "DE", "TD"))
def _run(table, idx, ts, W_t, b_t, B, L, DE, TD):
    info = plsc.get_sparse_core_info()
    NC, NS = info.num_cores, info.num_subcores
    NW = NC * NS
    D = DE + TD
    NB = B // NW
    C0, C1 = 104, L - 104

    mesh = plsc.VectorSubcoreMesh(core_axis_name="c", subcore_axis_name="s")

    @functools.partial(
        pl.kernel,
        mesh=mesh,
        out_type=jax.ShapeDtypeStruct((B * L, D), jnp.float32),
        scratch_types=[
            pltpu.VMEM((L,), jnp.int32),
            pltpu.VMEM((L,), jnp.float32),
            pltpu.VMEM((L, DE), jnp.float32),
            pltpu.VMEM((L, TD), jnp.float32),
            pltpu.VMEM((2, TD), jnp.float32),
            pltpu.VMEM((TD,), jnp.float32),
            pltpu.SemaphoreType.DMA,
        ],
        compiler_params=pltpu.CompilerParams(use_tc_tiling_on_sc=False),
    )
    def k(table_hbm, idx_hbm, ts_hbm, w_hbm, bt_hbm, out_hbm,
          idx_v, ts_v, emb_v, te_v, w_v, bt_v, gsem):
        wid = lax.axis_index("s") * NC + lax.axis_index("c")
        pltpu.sync_copy(w_hbm, w_v)
        pltpu.sync_copy(bt_hbm, bt_v)
        w1cm = w_v[1, :] * (1.0 / _SEC_PER_MONTH)
        wa = w_v[0, :] * (1.0 / _SEC_PER_YEAR) + w1cm
        bt = bt_v[...]
        zero = jnp.zeros((TD,), jnp.float32)

        def b_body(i, carry):
            base = (wid * NB + i) * L
            pltpu.sync_copy(idx_hbm.at[pl.ds(base, L)], idx_v)
            pltpu.sync_copy(ts_hbm.at[pl.ds(base, L)], ts_v)
            g0 = pltpu.async_copy(
                table_hbm.at[idx_v.at[pl.ds(0, C0)]],
                emb_v.at[pl.ds(0, C0)], gsem)
            g1 = pltpu.async_copy(
                table_hbm.at[idx_v.at[pl.ds(C0, C1)]],
                emb_v.at[pl.ds(C0, C1)], gsem)
            t0 = ts_v[pl.ds(0, 16)][0]
            btp = bt - t0 * w1cm

            def te_body(r, carry2):
                rr = r * 16
                tsv = ts_v[pl.ds(rr, 16)]
                for u in range(16):
                    te_v[rr + u, :] = jnp.maximum(tsv[u] * wa + btp, zero)
                return carry2

            lax.fori_loop(0, L // 16, te_body, 0)
            rr = (L // 16) * 16
            tsv = ts_v[pl.ds(L - 16, 16)]
            for u in range(rr, L):
                te_v[u, :] = jnp.maximum(tsv[u - (L - 16)] * wa + btp, zero)
            g0.wait()
            g1.wait()
            pltpu.sync_copy(emb_v, out_hbm.at[pl.ds(base, L), pl.ds(0, DE)])
            pltpu.sync_copy(te_v, out_hbm.at[pl.ds(base, L), pl.ds(DE, TD)])
            return carry

        lax.fori_loop(0, NB, b_body, 0)

    return k(table, idx, ts, W_t, b_t)


def kernel(indices, timestamps, table, W_t, b_t):
    B, L = indices.shape
    DE = table.shape[1]
    TD = b_t.shape[0]
    idx = indices.reshape(-1).astype(jnp.int32)
    ts = timestamps.reshape(-1)
    out = _run(table, idx, ts, W_t, b_t, B, L, DE, TD)
    return out.reshape(B, L, DE + TD)


# speedup vs baseline: 6.3458x; 1.3774x over previous
import functools

import jax
import jax.numpy as jnp
from jax import lax
from jax.experimental import pallas as pl
from jax.experimental.pallas import tpu as pltpu
from jax.experimental.pallas import tpu_sc as plsc

_SEC_PER_YEAR = 3600.0 * 24.0 * 365.0
_SEC_PER_MONTH = 3600.0 * 24.0 * 30.0


@functools.partial(jax.jit, static_argnames=("B", "L", "DE", "TD"))
def _run(table, idx, ts, W_t, b_t, B, L, DE, TD):
    info = plsc.get_sparse_core_info()
    NC, NS = info.num_cores, info.num_subcores
    NW = NC * NS
    D = DE + TD
    NB = B // NW
    NL = NB * L
    C0, C1 = 104, L - 104

    mesh = plsc.VectorSubcoreMesh(core_axis_name="c", subcore_axis_name="s")

    @functools.partial(
        pl.kernel,
        mesh=mesh,
        out_type=jax.ShapeDtypeStruct((B * L, D), jnp.float32),
        scratch_types=[
            pltpu.VMEM((NL,), jnp.int32),
            pltpu.VMEM((NL,), jnp.float32),
            pltpu.VMEM((2, L, DE), jnp.float32),
            pltpu.VMEM((2, L, TD), jnp.float32),
            pltpu.VMEM((2, TD), jnp.float32),
            pltpu.VMEM((TD,), jnp.float32),
            pltpu.SemaphoreType.DMA,
            pltpu.SemaphoreType.DMA,
            pltpu.SemaphoreType.DMA,
            pltpu.SemaphoreType.DMA,
        ],
        compiler_params=pltpu.CompilerParams(use_tc_tiling_on_sc=False),
    )
    def k(table_hbm, idx_hbm, ts_hbm, w_hbm, bt_hbm, out_hbm,
          idx_v, ts_v, emb_v, te_v, w_v, bt_v, gsemA, gsemB, osemA, osemB):
        wid = lax.axis_index("s") * NC + lax.axis_index("c")
        wbase = wid * NL
        pltpu.sync_copy(w_hbm, w_v)
        pltpu.sync_copy(bt_hbm, bt_v)
        pltpu.sync_copy(idx_hbm.at[pl.ds(wbase, NL)], idx_v)
        pltpu.sync_copy(ts_hbm.at[pl.ds(wbase, NL)], ts_v)
        w1cm = w_v[1, :] * (1.0 / _SEC_PER_MONTH)
        wa = w_v[0, :] * (1.0 / _SEC_PER_YEAR) + w1cm
        bt = bt_v[...]
        zero = jnp.zeros((TD,), jnp.float32)
        gsems = (gsemA, gsemB)
        osems = (osemA, osemB)

        def gathers(i, s, sem):
            off = i * L
            g0 = pltpu.async_copy(
                table_hbm.at[idx_v.at[pl.ds(off, C0)]],
                emb_v.at[s, pl.ds(0, C0)], sem)
            g1 = pltpu.async_copy(
                table_hbm.at[idx_v.at[pl.ds(off + C0, C1)]],
                emb_v.at[s, pl.ds(C0, C1)], sem)
            return g0, g1

        def out_copies(i, s, sem, issue):
            base = wbase + i * L
            mk = pltpu.async_copy if issue else pltpu.make_async_copy
            oe = mk(emb_v.at[s],
                    out_hbm.at[pl.ds(base, L), pl.ds(0, DE)], sem)
            ot = mk(te_v.at[s],
                    out_hbm.at[pl.ds(base, L), pl.ds(DE, TD)], sem)
            return oe, ot

        def te_compute(i, s):
            off = i * L
            t0 = ts_v[pl.ds(off, 16)][0]
            btp = bt - t0 * w1cm

            def te_body(r, carry):
                rr = r * 16
                tsv = ts_v[pl.ds(off + rr, 16)]
                for u in range(16):
                    te_v[s, rr + u, :] = jnp.maximum(tsv[u] * wa + btp, zero)
                return carry

            lax.fori_loop(0, L // 16, te_body, 0)
            rr = (L // 16) * 16
            tsv = ts_v[pl.ds(off + L - 16, 16)]
            for u in range(rr, L):
                te_v[s, u, :] = jnp.maximum(tsv[u - (L - 16)] * wa + btp, zero)

        def drain(s):
            oe, ot = out_copies(0, s, osems[s], issue=False)
            oe.wait()
            ot.wait()

        def j_body(j, carry):
            i0 = 2 * j
            i1 = i0 + 1

            @pl.when(j > 0)
            def _():
                drain(0)

            gA0, gA1 = gathers(i0, 0, gsems[0])

            @pl.when(j > 0)
            def _():
                drain(1)

            gB0, gB1 = gathers(i1, 1, gsems[1])
            te_compute(i0, 0)
            gA0.wait()
            gA1.wait()
            out_copies(i0, 0, osems[0], issue=True)
            te_compute(i1, 1)
            gB0.wait()
            gB1.wait()
            out_copies(i1, 1, osems[1], issue=True)
            return carry

        lax.fori_loop(0, NB // 2, j_body, 0)
        drain(0)
        drain(1)

    return k(table, idx, ts, W_t, b_t)


def kernel(indices, timestamps, table, W_t, b_t):
    B, L = indices.shape
    DE = table.shape[1]
    TD = b_t.shape[0]
    idx = indices.reshape(-1).astype(jnp.int32)
    ts = timestamps.reshape(-1)
    out = _run(table, idx, ts, W_t, b_t, B, L, DE, TD)
    return out.reshape(B, L, DE + TD)


# speedup vs baseline: 6.5582x; 1.0335x over previous
import functools

import jax
import jax.numpy as jnp
from jax import lax
from jax.experimental import pallas as pl
from jax.experimental.pallas import tpu as pltpu
from jax.experimental.pallas import tpu_sc as plsc

_SEC_PER_YEAR = 3600.0 * 24.0 * 365.0
_SEC_PER_MONTH = 3600.0 * 24.0 * 30.0


@functools.partial(jax.jit, static_argnames=("B", "L", "DE", "TD"))
def _run(table, idx, ts, W_t, b_t, B, L, DE, TD):
    info = plsc.get_sparse_core_info()
    NC, NS = info.num_cores, info.num_subcores
    NW = NC * NS
    D = DE + TD
    NB = B // NW
    NL = NB * L
    C0, C1 = 104, L - 104

    table_p = jnp.concatenate(
        [table, jnp.zeros((table.shape[0], TD), jnp.float32)], axis=1)
    wvec = jnp.concatenate([W_t[0], W_t[1], b_t])

    mesh = plsc.VectorSubcoreMesh(core_axis_name="c", subcore_axis_name="s")

    @functools.partial(
        pl.kernel,
        mesh=mesh,
        out_type=jax.ShapeDtypeStruct((B * L, D), jnp.float32),
        scratch_types=[
            pltpu.VMEM((NL,), jnp.int32),
            pltpu.VMEM((NL,), jnp.float32),
            pltpu.VMEM((2, L, D), jnp.float32),
            pltpu.VMEM((3 * TD,), jnp.float32),
            pltpu.SemaphoreType.DMA,
            pltpu.SemaphoreType.DMA,
            pltpu.SemaphoreType.DMA,
            pltpu.SemaphoreType.DMA,
        ],
        compiler_params=pltpu.CompilerParams(use_tc_tiling_on_sc=True),
    )
    def k(table_hbm, idx_hbm, ts_hbm, wvec_hbm, out_hbm,
          idx_v, ts_v, buf_v, wv, gsemA, gsemB, osemA, osemB):
        wid = lax.axis_index("s") * NC + lax.axis_index("c")
        wbase = wid * NL
        pltpu.sync_copy(wvec_hbm, wv)
        pltpu.sync_copy(idx_hbm.at[pl.ds(wbase, NL)], idx_v)
        pltpu.sync_copy(ts_hbm.at[pl.ds(wbase, NL)], ts_v)
        w1cm = wv[pl.ds(TD, TD)] * (1.0 / _SEC_PER_MONTH)
        wa = wv[pl.ds(0, TD)] * (1.0 / _SEC_PER_YEAR) + w1cm
        bt = wv[pl.ds(2 * TD, TD)]
        zero = jnp.zeros((TD,), jnp.float32)
        gsems = (gsemA, gsemB)
        osems = (osemA, osemB)

        def gathers(i, s, sem):
            off = i * L
            g0 = pltpu.async_copy(
                table_hbm.at[idx_v.at[pl.ds(off, C0)]],
                buf_v.at[s, pl.ds(0, C0)], sem)
            g1 = pltpu.async_copy(
                table_hbm.at[idx_v.at[pl.ds(off + C0, C1)]],
                buf_v.at[s, pl.ds(C0, C1)], sem)
            return g0, g1

        def out_copy(i, s, issue):
            base = wbase + i * L
            mk = pltpu.async_copy if issue else pltpu.make_async_copy
            return mk(buf_v.at[s], out_hbm.at[pl.ds(base, L)], osems[s])

        def te_compute(i, s):
            off = i * L
            t0 = ts_v[pl.ds(off, 16)][0]
            btp = bt - t0 * w1cm

            def te_body(r, carry):
                rr = r * 16
                tsv = ts_v[pl.ds(off + rr, 16)]
                for u in range(16):
                    buf_v[s, rr + u, pl.ds(DE, TD)] = jnp.maximum(
                        tsv[u] * wa + btp, zero)
                return carry

            lax.fori_loop(0, L // 16, te_body, 0)
            rr = (L // 16) * 16
            tsv = ts_v[pl.ds(off + L - 16, 16)]
            for u in range(rr, L):
                buf_v[s, u, pl.ds(DE, TD)] = jnp.maximum(
                    tsv[u - (L - 16)] * wa + btp, zero)

        def j_body(j, carry):
            i0 = 2 * j
            i1 = i0 + 1

            @pl.when(j > 0)
            def _():
                out_copy(0, 0, issue=False).wait()

            gA0, gA1 = gathers(i0, 0, gsems[0])

            @pl.when(j > 0)
            def _():
                out_copy(0, 1, issue=False).wait()

            gB0, gB1 = gathers(i1, 1, gsems[1])
            gA0.wait()
            gA1.wait()
            te_compute(i0, 0)
            out_copy(i0, 0, issue=True)
            gB0.wait()
            gB1.wait()
            te_compute(i1, 1)
            out_copy(i1, 1, issue=True)
            return carry

        lax.fori_loop(0, NB // 2, j_body, 0)
        out_copy(0, 0, issue=False).wait()
        out_copy(0, 1, issue=False).wait()

    return k(table_p, idx, ts, wvec)


def kernel(indices, timestamps, table, W_t, b_t):
    B, L = indices.shape
    DE = table.shape[1]
    TD = b_t.shape[0]
    idx = indices.reshape(-1).astype(jnp.int32)
    ts = timestamps.reshape(-1)
    out = _run(table, idx, ts, W_t, b_t, B, L, DE, TD)
    return out.reshape(B, L, DE + TD)


# speedup vs baseline: 7.5464x; 1.1507x over previous
import functools

import jax
import jax.numpy as jnp
from jax import lax
from jax.experimental import pallas as pl
from jax.experimental.pallas import tpu as pltpu
from jax.experimental.pallas import tpu_sc as plsc

_SEC_PER_YEAR = 3600.0 * 24.0 * 365.0
_SEC_PER_MONTH = 3600.0 * 24.0 * 30.0


def _transpose_pad(tT, TD):
    DE, V = tT.shape
    D = DE + TD
    BLK = 512
    grid = (V + BLK - 1) // BLK

    def body(in_ref, out_ref):
        x = in_ref[...]
        out_ref[:, 0:DE] = jnp.transpose(x)
        out_ref[:, DE:D] = jnp.zeros((BLK, TD), jnp.float32)

    return pl.pallas_call(
        body,
        grid=(grid,),
        in_specs=[pl.BlockSpec((DE, BLK), lambda i: (0, i))],
        out_specs=pl.BlockSpec((BLK, D), lambda i: (i, 0)),
        out_shape=jax.ShapeDtypeStruct((V, D), jnp.float32),
    )(tT)


@functools.partial(jax.jit, static_argnames=("B", "L", "DE", "TD"))
def _run(table, idx, ts, W_t, b_t, B, L, DE, TD):
    info = plsc.get_sparse_core_info()
    NC, NS = info.num_cores, info.num_subcores
    NW = NC * NS
    D = DE + TD
    NB = B // NW
    NL = NB * L
    C0, C1 = 104, L - 104

    table_p = _transpose_pad(jnp.transpose(table), TD)
    wvec = jnp.concatenate([W_t[0], W_t[1], b_t])

    mesh = plsc.VectorSubcoreMesh(core_axis_name="c", subcore_axis_name="s")

    @functools.partial(
        pl.kernel,
        mesh=mesh,
        out_type=jax.ShapeDtypeStruct((B * L, D), jnp.float32),
        scratch_types=[
            pltpu.VMEM((NL,), jnp.int32),
            pltpu.VMEM((NL,), jnp.float32),
            pltpu.VMEM((2, L, D), jnp.float32),
            pltpu.VMEM((3 * TD,), jnp.float32),
            pltpu.SemaphoreType.DMA,
            pltpu.SemaphoreType.DMA,
            pltpu.SemaphoreType.DMA,
            pltpu.SemaphoreType.DMA,
        ],
        compiler_params=pltpu.CompilerParams(use_tc_tiling_on_sc=True),
    )
    def k(table_hbm, idx_hbm, ts_hbm, wvec_hbm, out_hbm,
          idx_v, ts_v, buf_v, wv, gsemA, gsemB, osemA, osemB):
        wid = lax.axis_index("s") * NC + lax.axis_index("c")
        wbase = wid * NL
        pltpu.sync_copy(wvec_hbm, wv)
        pltpu.sync_copy(idx_hbm.at[pl.ds(wbase, NL)], idx_v)
        pltpu.sync_copy(ts_hbm.at[pl.ds(wbase, NL)], ts_v)
        w1cm = wv[pl.ds(TD, TD)] * (1.0 / _SEC_PER_MONTH)
        wa = wv[pl.ds(0, TD)] * (1.0 / _SEC_PER_YEAR) + w1cm
        bt = wv[pl.ds(2 * TD, TD)]
        zero = jnp.zeros((TD,), jnp.float32)
        gsems = (gsemA, gsemB)
        osems = (osemA, osemB)

        def gathers(i, s, sem):
            off = i * L
            g0 = pltpu.async_copy(
                table_hbm.at[idx_v.at[pl.ds(off, C0)]],
                buf_v.at[s, pl.ds(0, C0)], sem)
            g1 = pltpu.async_copy(
                table_hbm.at[idx_v.at[pl.ds(off + C0, C1)]],
                buf_v.at[s, pl.ds(C0, C1)], sem)
            return g0, g1

        def out_copy(i, s, issue):
            base = wbase + i * L
            mk = pltpu.async_copy if issue else pltpu.make_async_copy
            return mk(buf_v.at[s], out_hbm.at[pl.ds(base, L)], osems[s])

        def te_compute(i, s):
            off = i * L
            t0 = ts_v[pl.ds(off, 16)][0]
            btp = bt - t0 * w1cm

            def te_body(r, carry):
                rr = r * 16
                tsv = ts_v[pl.ds(off + rr, 16)]
                for u in range(16):
                    buf_v[s, rr + u, pl.ds(DE, TD)] = jnp.maximum(
                        tsv[u] * wa + btp, zero)
                return carry

            lax.fori_loop(0, L // 16, te_body, 0)
            rr = (L // 16) * 16
            tsv = ts_v[pl.ds(off + L - 16, 16)]
            for u in range(rr, L):
                buf_v[s, u, pl.ds(DE, TD)] = jnp.maximum(
                    tsv[u - (L - 16)] * wa + btp, zero)

        def j_body(j, carry):
            i0 = 2 * j
            i1 = i0 + 1

            @pl.when(j > 0)
            def _():
                out_copy(0, 0, issue=False).wait()

            gA0, gA1 = gathers(i0, 0, gsems[0])

            @pl.when(j > 0)
            def _():
                out_copy(0, 1, issue=False).wait()

            gB0, gB1 = gathers(i1, 1, gsems[1])
            gA0.wait()
            gA1.wait()
            te_compute(i0, 0)
            out_copy(i0, 0, issue=True)
            gB0.wait()
            gB1.wait()
            te_compute(i1, 1)
            out_copy(i1, 1, issue=True)
            return carry

        lax.fori_loop(0, NB // 2, j_body, 0)
        out_copy(0, 0, issue=False).wait()
        out_copy(0, 1, issue=False).wait()

    return k(table_p, idx, ts, wvec)


def kernel(indices, timestamps, table, W_t, b_t):
    B, L = indices.shape
    DE = table.shape[1]
    TD = b_t.shape[0]
    idx = indices.reshape(-1).astype(jnp.int32)
    ts = timestamps.reshape(-1)
    out = _run(table, idx, ts, W_t, b_t, B, L, DE, TD)
    return out.reshape(B, L, DE + TD)


# speedup vs baseline: 7.6202x; 1.0098x over previous
import functools

import jax
import jax.numpy as jnp
from jax import lax
from jax.experimental import pallas as pl
from jax.experimental.pallas import tpu as pltpu
from jax.experimental.pallas import tpu_sc as plsc

_SEC_PER_YEAR = 3600.0 * 24.0 * 365.0
_SEC_PER_MONTH = 3600.0 * 24.0 * 30.0


def _transpose_pad(tT, TD):
    DE, V = tT.shape
    D = DE + TD
    BLK = 512
    grid = (V + BLK - 1) // BLK

    def body(in_ref, out_ref):
        x = in_ref[...]
        out_ref[:, 0:DE] = jnp.transpose(x)
        out_ref[:, DE:D] = jnp.zeros((BLK, TD), jnp.float32)

    return pl.pallas_call(
        body,
        grid=(grid,),
        in_specs=[pl.BlockSpec((DE, BLK), lambda i: (0, i))],
        out_specs=pl.BlockSpec((BLK, D), lambda i: (i, 0)),
        out_shape=jax.ShapeDtypeStruct((V, D), jnp.float32),
    )(tT)


@functools.partial(jax.jit, static_argnames=("B", "L", "DE", "TD"))
def _run(table, idx, ts, W_t, b_t, B, L, DE, TD):
    info = plsc.get_sparse_core_info()
    NC, NS = info.num_cores, info.num_subcores
    NW = NC * NS
    D = DE + TD
    NB = B // NW
    NL = NB * L
    C0, C1 = 104, L - 104

    table_p = _transpose_pad(jnp.transpose(table), TD)
    wvec = jnp.concatenate([W_t[0], W_t[1], b_t])

    mesh = plsc.VectorSubcoreMesh(core_axis_name="c", subcore_axis_name="s")

    @functools.partial(
        pl.kernel,
        mesh=mesh,
        out_type=jax.ShapeDtypeStruct((B * L, D), jnp.float32),
        scratch_types=[
            pltpu.VMEM((NL,), jnp.int32),
            pltpu.VMEM((NL,), jnp.float32),
            pltpu.VMEM((2, L, D), jnp.float32),
            pltpu.VMEM((L, TD), jnp.float32),
            pltpu.VMEM((3 * TD,), jnp.float32),
            pltpu.SemaphoreType.DMA,
            pltpu.SemaphoreType.DMA,
            pltpu.SemaphoreType.DMA,
            pltpu.SemaphoreType.DMA,
        ],
        compiler_params=pltpu.CompilerParams(use_tc_tiling_on_sc=True),
    )
    def k(table_hbm, idx_hbm, ts_hbm, wvec_hbm, out_hbm,
          idx_v, ts_v, buf_v, te_v, wv, gsemA, gsemB, osemA, osemB):
        wid = lax.axis_index("s") * NC + lax.axis_index("c")
        wbase = wid * NL
        pltpu.sync_copy(wvec_hbm, wv)
        pltpu.sync_copy(idx_hbm.at[pl.ds(wbase, NL)], idx_v)
        pltpu.sync_copy(ts_hbm.at[pl.ds(wbase, NL)], ts_v)
        w1cm = wv[pl.ds(TD, TD)] * (1.0 / _SEC_PER_MONTH)
        wa = wv[pl.ds(0, TD)] * (1.0 / _SEC_PER_YEAR) + w1cm
        bt = wv[pl.ds(2 * TD, TD)]
        zero = jnp.zeros((TD,), jnp.float32)
        gsems = (gsemA, gsemB)
        osems = (osemA, osemB)

        def gathers(i, s, sem):
            off = i * L
            g0 = pltpu.async_copy(
                table_hbm.at[idx_v.at[pl.ds(off, C0)]],
                buf_v.at[s, pl.ds(0, C0)], sem)
            g1 = pltpu.async_copy(
                table_hbm.at[idx_v.at[pl.ds(off + C0, C1)]],
                buf_v.at[s, pl.ds(C0, C1)], sem)
            return g0, g1

        def out_copy(i, s, issue):
            base = wbase + i * L
            mk = pltpu.async_copy if issue else pltpu.make_async_copy
            return mk(buf_v.at[s], out_hbm.at[pl.ds(base, L)], osems[s])

        def te_compute(i, s):
            off = i * L
            t0 = ts_v[pl.ds(off, 16)][0]
            btp = bt - t0 * w1cm

            def te_body(r, carry):
                rr = r * 16
                tsv = ts_v[pl.ds(off + rr, 16)]
                for u in range(16):
                    te_v[rr + u, :] = jnp.maximum(tsv[u] * wa + btp, zero)
                return carry

            lax.fori_loop(0, L // 16, te_body, 0)
            rr = (L // 16) * 16
            tsv = ts_v[pl.ds(off + L - 16, 16)]
            for u in range(rr, L):
                te_v[u, :] = jnp.maximum(
                    tsv[u - (L - 16)] * wa + btp, zero)

        def te_blit(s):
            def blit_body(r, carry):
                rr = r * 8
                for u in range(8):
                    buf_v[s, rr + u, pl.ds(DE, TD)] = te_v[rr + u, :]
                return carry

            lax.fori_loop(0, L // 8, blit_body, 0)

        def j_body(j, carry):
            i0 = 2 * j
            i1 = i0 + 1

            @pl.when(j > 0)
            def _():
                out_copy(0, 0, issue=False).wait()

            gA0, gA1 = gathers(i0, 0, gsems[0])

            @pl.when(j > 0)
            def _():
                out_copy(0, 1, issue=False).wait()

            gB0, gB1 = gathers(i1, 1, gsems[1])
            te_compute(i0, 0)
            gA0.wait()
            gA1.wait()
            te_blit(0)
            out_copy(i0, 0, issue=True)
            te_compute(i1, 1)
            gB0.wait()
            gB1.wait()
            te_blit(1)
            out_copy(i1, 1, issue=True)
            return carry

        lax.fori_loop(0, NB // 2, j_body, 0)
        out_copy(0, 0, issue=False).wait()
        out_copy(0, 1, issue=False).wait()

    return k(table_p, idx, ts, wvec)


def kernel(indices, timestamps, table, W_t, b_t):
    B, L = indices.shape
    DE = table.shape[1]
    TD = b_t.shape[0]
    idx = indices.reshape(-1).astype(jnp.int32)
    ts = timestamps.reshape(-1)
    out = _run(table, idx, ts, W_t, b_t, B, L, DE, TD)
    return out.reshape(B, L, DE + TD)


# speedup vs baseline: 8.9858x; 1.1792x over previous
import functools

import jax
import jax.numpy as jnp
from jax import lax
from jax.experimental import pallas as pl
from jax.experimental.pallas import tpu as pltpu
from jax.experimental.pallas import tpu_sc as plsc

_SEC_PER_YEAR = 3600.0 * 24.0 * 365.0
_SEC_PER_MONTH = 3600.0 * 24.0 * 30.0


def _transpose_pad(tT, TD):
    DE, V = tT.shape
    D = DE + TD
    BLK = 2048
    grid = (V + BLK - 1) // BLK

    def body(in_ref, out_ref):
        x = in_ref[...]
        out_ref[:, 0:DE] = jnp.transpose(x)
        out_ref[:, DE:D] = jnp.zeros((BLK, TD), jnp.float32)

    return pl.pallas_call(
        body,
        grid=(grid,),
        in_specs=[pl.BlockSpec((DE, BLK), lambda i: (0, i))],
        out_specs=pl.BlockSpec((BLK, D), lambda i: (i, 0)),
        out_shape=jax.ShapeDtypeStruct((V, D), jnp.float32),
    )(tT)


@functools.partial(jax.jit, static_argnames=("B", "L", "DE", "TD"))
def _run(table, idx, ts, W_t, b_t, B, L, DE, TD):
    info = plsc.get_sparse_core_info()
    NC, NS = info.num_cores, info.num_subcores
    NW = NC * NS
    D = DE + TD
    NB = B // NW
    NL = NB * L
    C0, C1 = 104, L - 104

    table_p = _transpose_pad(jnp.transpose(table), TD)
    wvec = jnp.concatenate([W_t[0], W_t[1], b_t])

    mesh = plsc.VectorSubcoreMesh(core_axis_name="c", subcore_axis_name="s")

    @functools.partial(
        pl.kernel,
        mesh=mesh,
        out_type=jax.ShapeDtypeStruct((B * L, D), jnp.float32),
        scratch_types=[
            pltpu.VMEM((NL,), jnp.int32),
            pltpu.VMEM((NL,), jnp.float32),
            pltpu.VMEM((2, L, D), jnp.float32),
            pltpu.VMEM((L, TD), jnp.float32),
            pltpu.VMEM((3 * TD,), jnp.float32),
            pltpu.SemaphoreType.DMA,
            pltpu.SemaphoreType.DMA,
            pltpu.SemaphoreType.DMA,
            pltpu.SemaphoreType.DMA,
        ],
        compiler_params=pltpu.CompilerParams(use_tc_tiling_on_sc=True),
    )
    def k(table_hbm, idx_hbm, ts_hbm, wvec_hbm, out_hbm,
          idx_v, ts_v, buf_v, te_v, wv, gsemA, gsemB, osemA, osemB):
        wid = lax.axis_index("s") * NC + lax.axis_index("c")
        wbase = wid * NL
        pltpu.sync_copy(wvec_hbm, wv)
        pltpu.sync_copy(idx_hbm.at[pl.ds(wbase, NL)], idx_v)
        pltpu.sync_copy(ts_hbm.at[pl.ds(wbase, NL)], ts_v)
        w1cm = wv[pl.ds(TD, TD)] * (1.0 / _SEC_PER_MONTH)
        wa = wv[pl.ds(0, TD)] * (1.0 / _SEC_PER_YEAR) + w1cm
        bt = wv[pl.ds(2 * TD, TD)]
        zero = jnp.zeros((TD,), jnp.float32)
        gsems = (gsemA, gsemB)
        osems = (osemA, osemB)

        def gathers(i, s, sem):
            off = i * L
            g0 = pltpu.async_copy(
                table_hbm.at[idx_v.at[pl.ds(off, C0)]],
                buf_v.at[s, pl.ds(0, C0)], sem)
            g1 = pltpu.async_copy(
                table_hbm.at[idx_v.at[pl.ds(off + C0, C1)]],
                buf_v.at[s, pl.ds(C0, C1)], sem)
            return g0, g1

        def out_copy(i, s, issue):
            base = wbase + i * L
            mk = pltpu.async_copy if issue else pltpu.make_async_copy
            return mk(buf_v.at[s], out_hbm.at[pl.ds(base, L)], osems[s])

        def te_compute(i, s):
            off = i * L
            t0 = ts_v[pl.ds(off, 16)][0]
            btp = bt - t0 * w1cm

            def te_body(r, carry):
                rr = r * 16
                tsv = ts_v[pl.ds(off + rr, 16)]
                for u in range(16):
                    te_v[rr + u, :] = jnp.maximum(tsv[u] * wa + btp, zero)
                return carry

            lax.fori_loop(0, L // 16, te_body, 0)
            rr = (L // 16) * 16
            tsv = ts_v[pl.ds(off + L - 16, 16)]
            for u in range(rr, L):
                te_v[u, :] = jnp.maximum(
                    tsv[u - (L - 16)] * wa + btp, zero)

        def te_blit(s):
            def blit_body(r, carry):
                rr = r * 8
                for u in range(8):
                    buf_v[s, rr + u, pl.ds(DE, TD)] = te_v[rr + u, :]
                return carry

            lax.fori_loop(0, L // 8, blit_body, 0)

        def j_body(j, carry):
            i0 = 2 * j
            i1 = i0 + 1

            @pl.when(j > 0)
            def _():
                out_copy(0, 0, issue=False).wait()

            gA0, gA1 = gathers(i0, 0, gsems[0])

            @pl.when(j > 0)
            def _():
                out_copy(0, 1, issue=False).wait()

            gB0, gB1 = gathers(i1, 1, gsems[1])
            te_compute(i0, 0)
            gA0.wait()
            gA1.wait()
            te_blit(0)
            out_copy(i0, 0, issue=True)
            te_compute(i1, 1)
            gB0.wait()
            gB1.wait()
            te_blit(1)
            out_copy(i1, 1, issue=True)
            return carry

        lax.fori_loop(0, NB // 2, j_body, 0)
        out_copy(0, 0, issue=False).wait()
        out_copy(0, 1, issue=False).wait()

    return k(table_p, idx, ts, wvec)


def kernel(indices, timestamps, table, W_t, b_t):
    B, L = indices.shape
    DE = table.shape[1]
    TD = b_t.shape[0]
    idx = indices.reshape(-1).astype(jnp.int32)
    ts = timestamps.reshape(-1)
    out = _run(table, idx, ts, W_t, b_t, B, L, DE, TD)
    return out.reshape(B, L, DE + TD)


# speedup vs baseline: 9.3496x; 1.0405x over previous
import functools

import jax
import jax.numpy as jnp
from jax import lax
from jax.experimental import pallas as pl
from jax.experimental.pallas import tpu as pltpu
from jax.experimental.pallas import tpu_sc as plsc

_SEC_PER_YEAR = 3600.0 * 24.0 * 365.0
_SEC_PER_MONTH = 3600.0 * 24.0 * 30.0


def _transpose_pad(tT, TD):
    DE, V = tT.shape
    D = DE + TD
    BLK = 4096
    grid = (V + BLK - 1) // BLK

    def body(in_ref, out_ref):
        x = in_ref[...]
        out_ref[:, 0:DE] = jnp.transpose(x)
        out_ref[:, DE:D] = jnp.zeros((BLK, TD), jnp.float32)

    return pl.pallas_call(
        body,
        grid=(grid,),
        in_specs=[pl.BlockSpec((DE, BLK), lambda i: (0, i))],
        out_specs=pl.BlockSpec((BLK, D), lambda i: (i, 0)),
        out_shape=jax.ShapeDtypeStruct((V, D), jnp.float32),
    )(tT)


@functools.partial(jax.jit, static_argnames=("B", "L", "DE", "TD"))
def _run(table, idx, ts, W_t, b_t, B, L, DE, TD):
    info = plsc.get_sparse_core_info()
    NC, NS = info.num_cores, info.num_subcores
    NW = NC * NS
    D = DE + TD
    NB = B // NW
    NL = NB * L
    C0, C1 = 104, L - 104

    table_p = _transpose_pad(jnp.transpose(table), TD)
    wvec = jnp.concatenate([W_t[0], W_t[1], b_t])

    mesh = plsc.VectorSubcoreMesh(core_axis_name="c", subcore_axis_name="s")

    @functools.partial(
        pl.kernel,
        mesh=mesh,
        out_type=jax.ShapeDtypeStruct((B * L, D), jnp.float32),
        scratch_types=[
            pltpu.VMEM((NL,), jnp.int32),
            pltpu.VMEM((NL,), jnp.float32),
            pltpu.VMEM((2, L, D), jnp.float32),
            pltpu.VMEM((L, TD), jnp.float32),
            pltpu.VMEM((3 * TD,), jnp.float32),
            pltpu.SemaphoreType.DMA,
            pltpu.SemaphoreType.DMA,
            pltpu.SemaphoreType.DMA,
            pltpu.SemaphoreType.DMA,
        ],
        compiler_params=pltpu.CompilerParams(use_tc_tiling_on_sc=True),
    )
    def k(table_hbm, idx_hbm, ts_hbm, wvec_hbm, out_hbm,
          idx_v, ts_v, buf_v, te_v, wv, gsemA, gsemB, osemA, osemB):
        wid = lax.axis_index("s") * NC + lax.axis_index("c")
        wbase = wid * NL
        pltpu.sync_copy(wvec_hbm, wv)
        pltpu.sync_copy(idx_hbm.at[pl.ds(wbase, NL)], idx_v)
        pltpu.sync_copy(ts_hbm.at[pl.ds(wbase, NL)], ts_v)
        w1cm = wv[pl.ds(TD, TD)] * (1.0 / _SEC_PER_MONTH)
        wa = wv[pl.ds(0, TD)] * (1.0 / _SEC_PER_YEAR) + w1cm
        bt = wv[pl.ds(2 * TD, TD)]
        zero = jnp.zeros((TD,), jnp.float32)
        gsems = (gsemA, gsemB)
        osems = (osemA, osemB)

        def gathers(i, s, sem):
            off = i * L
            g0 = pltpu.async_copy(
                table_hbm.at[idx_v.at[pl.ds(off, C0)]],
                buf_v.at[s, pl.ds(0, C0)], sem)
            g1 = pltpu.async_copy(
                table_hbm.at[idx_v.at[pl.ds(off + C0, C1)]],
                buf_v.at[s, pl.ds(C0, C1)], sem)
            return g0, g1

        def out_copy(i, s, issue):
            base = wbase + i * L
            mk = pltpu.async_copy if issue else pltpu.make_async_copy
            return mk(buf_v.at[s], out_hbm.at[pl.ds(base, L)], osems[s])

        def te_compute(i, s):
            off = i * L
            t0 = ts_v[pl.ds(off, 16)][0]
            btp = bt - t0 * w1cm

            def te_body(r, carry):
                rr = r * 16
                tsv = ts_v[pl.ds(off + rr, 16)]
                for u in range(16):
                    te_v[rr + u, :] = jnp.maximum(tsv[u] * wa + btp, zero)
                return carry

            lax.fori_loop(0, L // 16, te_body, 0)
            rr = (L // 16) * 16
            tsv = ts_v[pl.ds(off + L - 16, 16)]
            for u in range(rr, L):
                te_v[u, :] = jnp.maximum(
                    tsv[u - (L - 16)] * wa + btp, zero)

        def te_blit(s):
            def blit_body(r, carry):
                rr = r * 8
                for u in range(8):
                    buf_v[s, rr + u, pl.ds(DE, TD)] = te_v[rr + u, :]
                return carry

            lax.fori_loop(0, L // 8, blit_body, 0)

        def j_body(j, carry):
            i0 = 2 * j
            i1 = i0 + 1

            @pl.when(j > 0)
            def _():
                out_copy(0, 0, issue=False).wait()

            gA0, gA1 = gathers(i0, 0, gsems[0])

            @pl.when(j > 0)
            def _():
                out_copy(0, 1, issue=False).wait()

            gB0, gB1 = gathers(i1, 1, gsems[1])
            te_compute(i0, 0)
            gA0.wait()
            gA1.wait()
            te_blit(0)
            out_copy(i0, 0, issue=True)
            te_compute(i1, 1)
            gB0.wait()
            gB1.wait()
            te_blit(1)
            out_copy(i1, 1, issue=True)
            return carry

        lax.fori_loop(0, NB // 2, j_body, 0)
        out_copy(0, 0, issue=False).wait()
        out_copy(0, 1, issue=False).wait()

    return k(table_p, idx, ts, wvec)


def kernel(indices, timestamps, table, W_t, b_t):
    B, L = indices.shape
    DE = table.shape[1]
    TD = b_t.shape[0]
    idx = indices.reshape(-1).astype(jnp.int32)
    ts = timestamps.reshape(-1)
    out = _run(table, idx, ts, W_t, b_t, B, L, DE, TD)
    return out.reshape(B, L, DE + TD)


# speedup vs baseline: 9.5225x; 1.0185x over previous
import functools

import jax
import jax.numpy as jnp
from jax import lax
from jax.experimental import pallas as pl
from jax.experimental.pallas import tpu as pltpu
from jax.experimental.pallas import tpu_sc as plsc

_SEC_PER_YEAR = 3600.0 * 24.0 * 365.0
_SEC_PER_MONTH = 3600.0 * 24.0 * 30.0


def _transpose_pad(tT, TD):
    DE, V = tT.shape
    D = DE + TD
    BLK = 8192
    grid = (V + BLK - 1) // BLK

    def body(in_ref, out_ref):
        x = in_ref[...]
        out_ref[:, 0:DE] = jnp.transpose(x)
        out_ref[:, DE:D] = jnp.zeros((BLK, TD), jnp.float32)

    return pl.pallas_call(
        body,
        grid=(grid,),
        in_specs=[pl.BlockSpec((DE, BLK), lambda i: (0, i))],
        out_specs=pl.BlockSpec((BLK, D), lambda i: (i, 0)),
        out_shape=jax.ShapeDtypeStruct((V, D), jnp.float32),
    )(tT)


@functools.partial(jax.jit, static_argnames=("B", "L", "DE", "TD"))
def _run(table, idx, ts, W_t, b_t, B, L, DE, TD):
    info = plsc.get_sparse_core_info()
    NC, NS = info.num_cores, info.num_subcores
    NW = NC * NS
    D = DE + TD
    NB = B // NW
    NL = NB * L
    C0, C1 = 104, L - 104

    table_p = _transpose_pad(jnp.transpose(table), TD)
    wvec = jnp.concatenate([W_t[0], W_t[1], b_t])

    mesh = plsc.VectorSubcoreMesh(core_axis_name="c", subcore_axis_name="s")

    @functools.partial(
        pl.kernel,
        mesh=mesh,
        out_type=jax.ShapeDtypeStruct((B * L, D), jnp.float32),
        scratch_types=[
            pltpu.VMEM((NL,), jnp.int32),
            pltpu.VMEM((NL,), jnp.float32),
            pltpu.VMEM((2, L, D), jnp.float32),
            pltpu.VMEM((L, TD), jnp.float32),
            pltpu.VMEM((3 * TD,), jnp.float32),
            pltpu.SemaphoreType.DMA,
            pltpu.SemaphoreType.DMA,
            pltpu.SemaphoreType.DMA,
            pltpu.SemaphoreType.DMA,
        ],
        compiler_params=pltpu.CompilerParams(use_tc_tiling_on_sc=True),
    )
    def k(table_hbm, idx_hbm, ts_hbm, wvec_hbm, out_hbm,
          idx_v, ts_v, buf_v, te_v, wv, gsemA, gsemB, osemA, osemB):
        wid = lax.axis_index("s") * NC + lax.axis_index("c")
        wbase = wid * NL
        pltpu.sync_copy(wvec_hbm, wv)
        pltpu.sync_copy(idx_hbm.at[pl.ds(wbase, NL)], idx_v)
        pltpu.sync_copy(ts_hbm.at[pl.ds(wbase, NL)], ts_v)
        w1cm = wv[pl.ds(TD, TD)] * (1.0 / _SEC_PER_MONTH)
        wa = wv[pl.ds(0, TD)] * (1.0 / _SEC_PER_YEAR) + w1cm
        bt = wv[pl.ds(2 * TD, TD)]
        zero = jnp.zeros((TD,), jnp.float32)
        gsems = (gsemA, gsemB)
        osems = (osemA, osemB)

        def gathers(i, s, sem):
            off = i * L
            g0 = pltpu.async_copy(
                table_hbm.at[idx_v.at[pl.ds(off, C0)]],
                buf_v.at[s, pl.ds(0, C0)], sem)
            g1 = pltpu.async_copy(
                table_hbm.at[idx_v.at[pl.ds(off + C0, C1)]],
                buf_v.at[s, pl.ds(C0, C1)], sem)
            return g0, g1

        def out_copy(i, s, issue):
            base = wbase + i * L
            mk = pltpu.async_copy if issue else pltpu.make_async_copy
            return mk(buf_v.at[s], out_hbm.at[pl.ds(base, L)], osems[s])

        def te_compute(i, s):
            off = i * L
            t0 = ts_v[pl.ds(off, 16)][0]
            btp = bt - t0 * w1cm

            def te_body(r, carry):
                rr = r * 16
                tsv = ts_v[pl.ds(off + rr, 16)]
                for u in range(16):
                    te_v[rr + u, :] = jnp.maximum(tsv[u] * wa + btp, zero)
                return carry

            lax.fori_loop(0, L // 16, te_body, 0)
            rr = (L // 16) * 16
            tsv = ts_v[pl.ds(off + L - 16, 16)]
            for u in range(rr, L):
                te_v[u, :] = jnp.maximum(
                    tsv[u - (L - 16)] * wa + btp, zero)

        def te_blit(s):
            def blit_body(r, carry):
                rr = r * 8
                for u in range(8):
                    buf_v[s, rr + u, pl.ds(DE, TD)] = te_v[rr + u, :]
                return carry

            lax.fori_loop(0, L // 8, blit_body, 0)

        def j_body(j, carry):
            i0 = 2 * j
            i1 = i0 + 1

            @pl.when(j > 0)
            def _():
                out_copy(0, 0, issue=False).wait()

            gA0, gA1 = gathers(i0, 0, gsems[0])

            @pl.when(j > 0)
            def _():
                out_copy(0, 1, issue=False).wait()

            gB0, gB1 = gathers(i1, 1, gsems[1])
            te_compute(i0, 0)
            gA0.wait()
            gA1.wait()
            te_blit(0)
            out_copy(i0, 0, issue=True)
            te_compute(i1, 1)
            gB0.wait()
            gB1.wait()
            te_blit(1)
            out_copy(i1, 1, issue=True)
            return carry

        lax.fori_loop(0, NB // 2, j_body, 0)
        out_copy(0, 0, issue=False).wait()
        out_copy(0, 1, issue=False).wait()

    return k(table_p, idx, ts, wvec)


def kernel(indices, timestamps, table, W_t, b_t):
    B, L = indices.shape
    DE = table.shape[1]
    TD = b_t.shape[0]
    idx = indices.reshape(-1).astype(jnp.int32)
    ts = timestamps.reshape(-1)
    out = _run(table, idx, ts, W_t, b_t, B, L, DE, TD)
    return out.reshape(B, L, DE + TD)
